# baseline (speedup 1.0000x reference)
import functools

import jax
import jax.numpy as jnp
from jax import lax
from jax.experimental import pallas as pl
from jax.experimental.pallas import tpu as pltpu
from jax.experimental.pallas import tpu_sc as plsc

K = 8192
D = 256
N = 8192
BT = 512
CHUNKS = ((0, 2048), (2048, 4096), (4096, 6144), (6144, 8192))
COMMITMENT_COST = 0.5
HOLISTIC_COST = 0.1


def _argmin_body(z2_ref, *refs):
    nch = len(CHUNKS)
    e_ref = refs[0]
    a_ref = refs[1]
    b_refs = refs[2:nch + 2]
    idx_ref, enc_ref, lsum_ref = refs[nch + 2:]
    t = pl.program_id(0)
    z2 = z2_ref[...]
    a = a_ref[...]
    w = CHUNKS[0][1] - CHUNKS[0][0]
    iota_s = lax.broadcasted_iota(jnp.int32, (BT, w), 1).astype(
        jnp.float32) * jnp.float32(2.0 ** -40)
    m_run = jnp.full((BT, 1), jnp.inf, jnp.float32)
    i_run = jnp.zeros((BT, 1), jnp.int32)
    for (off, end), b_ref in zip(CHUNKS, b_refs):
        c2 = lax.dot_general(z2, e_ref[off:end, :], (((1,), (1,)), ((), ())),
                             preferred_element_type=jnp.float32)
        d = (a + b_ref[...]) - c2
        cm = jnp.min(d, axis=1, keepdims=True)
        cand = (d - cm) + iota_s
        ci = (jnp.min(cand, axis=1, keepdims=True)
              * jnp.float32(2.0 ** 40)).astype(jnp.int32) + off
        upd = cm < m_run
        i_run = jnp.where(upd, ci, i_run)
        m_run = jnp.where(upd, cm, m_run).astype(jnp.bfloat16).astype(
            jnp.float32)
    idx_ref[...] = i_run
    col = lax.broadcasted_iota(jnp.int32, (BT, K), 1)
    enc_ref[...] = (col == i_run).astype(jnp.float32)
    s = jnp.sum(m_run, keepdims=True)

    @pl.when(t == 0)
    def _():
        lsum_ref[...] = s

    @pl.when(t != 0)
    def _():
        lsum_ref[...] += s


def _argmin_onehot(flat, e_ts, a, bs):
    grid = (N // BT,)
    widths = [e - o for o, e in CHUNKS]
    return pl.pallas_call(
        _argmin_body,
        grid=grid,
        in_specs=(
            [pl.BlockSpec((BT, D), lambda t: (t, 0)),
             pl.BlockSpec((K, D), lambda t: (0, 0)),
             pl.BlockSpec((BT, 1), lambda t: (t, 0))]
            + [pl.BlockSpec((1, w), lambda t: (0, 0)) for w in widths]
        ),
        out_specs=[
            pl.BlockSpec((BT, 1), lambda t: (t, 0)),
            pl.BlockSpec((BT, K), lambda t: (t, 0)),
            pl.BlockSpec((1, 1), lambda t: (0, 0)),
        ],
        out_shape=[
            jax.ShapeDtypeStruct((N, 1), jnp.int32),
            jax.ShapeDtypeStruct((N, K), jnp.float32),
            jax.ShapeDtypeStruct((1, 1), jnp.float32),
        ],
    )(flat, e_ts, a, *bs)


def _make_sc_gather():
    info = plsc.get_sparse_core_info()
    nw = info.num_cores * info.num_subcores
    b_per_w = N // nw
    chunk = 128
    nchunk = b_per_w // chunk
    mesh = plsc.VectorSubcoreMesh(core_axis_name="c", subcore_axis_name="s")

    @functools.partial(
        pl.kernel, mesh=mesh,
        out_type=jax.ShapeDtypeStruct((N, D), jnp.float32),
        scratch_types=[
            pltpu.VMEM((chunk,), jnp.int32),
            pltpu.VMEM((chunk, D), jnp.float32),
            pltpu.SemaphoreType.DMA,
        ],
    )
    def gather_k(table_hbm, idx_hbm, out_hbm, idx_v, rows_v, sem):
        wid = lax.axis_index("s") * info.num_cores + lax.axis_index("c")
        base = wid * b_per_w
        for cidx in range(nchunk):
            off = base + cidx * chunk
            pltpu.sync_copy(idx_hbm.at[pl.ds(off, chunk)], idx_v)
            pltpu.async_copy(table_hbm.at[idx_v], rows_v, sem).wait()
            pltpu.sync_copy(rows_v, out_hbm.at[pl.ds(off, chunk)])

    return gather_k


def kernel(inputs, embedding):
    input_shape = inputs.shape
    flat = inputs.reshape(-1, D)
    a = jnp.sum(flat ** 2, axis=1, keepdims=True)
    b = jnp.sum(embedding ** 2, axis=1)[None, :]
    bs = tuple(b[:, o:e] for o, e in CHUNKS)

    idx2d, encodings, lsum = _argmin_onehot(flat * 2.0, embedding, a, bs)

    gather_k = _make_sc_gather()
    quantized = gather_k(embedding, idx2d.reshape(-1))

    mse = lsum[0, 0] / jnp.float32(N * D)
    loss = HOLISTIC_COST * (mse + COMMITMENT_COST * mse)
    quantized_st = quantized.reshape(input_shape)
    return (loss, quantized_st, encodings, idx2d)


# speedup vs baseline: 1.0072x; 1.0072x over previous
import functools

import jax
import jax.numpy as jnp
from jax import lax
from jax.experimental import pallas as pl
from jax.experimental.pallas import tpu as pltpu
from jax.experimental.pallas import tpu_sc as plsc

K = 8192
D = 256
N = 8192
BT = 512
CHUNKS = ((0, 2048), (2048, 4096), (4096, 6144), (6144, 8192))
COMMITMENT_COST = 0.5
HOLISTIC_COST = 0.1


def _argmin_body(z2_ref, *refs):
    nch = len(CHUNKS)
    e_ref = refs[0]
    a_ref = refs[1]
    b_refs = refs[2:nch + 2]
    idx_ref, enc_ref, lsum_ref = refs[nch + 2:]
    t = pl.program_id(0)
    z2 = z2_ref[...]
    a = jnp.transpose(a_ref[0], (1, 0))
    w = CHUNKS[0][1] - CHUNKS[0][0]
    iota_s = lax.broadcasted_iota(jnp.int32, (BT, w), 1).astype(
        jnp.float32) * jnp.float32(2.0 ** -40)
    m_run = jnp.full((BT, 1), jnp.inf, jnp.float32)
    i_run = jnp.zeros((BT, 1), jnp.int32)
    for (off, end), b_ref in zip(CHUNKS, b_refs):
        c2 = lax.dot_general(z2, e_ref[off:end, :], (((1,), (1,)), ((), ())),
                             preferred_element_type=jnp.float32)
        d = (a + b_ref[...]) - c2
        cm = jnp.min(d, axis=1, keepdims=True)
        cand = (d - cm) + iota_s
        ci = (jnp.min(cand, axis=1, keepdims=True)
              * jnp.float32(2.0 ** 40)).astype(jnp.int32) + off
        upd = cm < m_run
        i_run = jnp.where(upd, ci, i_run)
        m_run = jnp.where(upd, cm, m_run).astype(jnp.bfloat16).astype(
            jnp.float32)
    idx_ref[...] = i_run
    col = lax.broadcasted_iota(jnp.int32, (BT, K), 1)
    enc_ref[...] = (col == i_run).astype(jnp.float32)
    s = jnp.sum(m_run, keepdims=True)

    @pl.when(t == 0)
    def _():
        lsum_ref[...] = s

    @pl.when(t != 0)
    def _():
        lsum_ref[...] += s


def _argmin_onehot(flat, e_ts, a, bs):
    grid = (N // BT,)
    widths = [e - o for o, e in CHUNKS]
    return pl.pallas_call(
        _argmin_body,
        grid=grid,
        in_specs=(
            [pl.BlockSpec((BT, D), lambda t: (t, 0)),
             pl.BlockSpec((K, D), lambda t: (0, 0)),
             pl.BlockSpec((1, 1, BT), lambda t: (t, 0, 0))]
            + [pl.BlockSpec((1, w), lambda t: (0, 0)) for w in widths]
        ),
        out_specs=[
            pl.BlockSpec((BT, 1), lambda t: (t, 0)),
            pl.BlockSpec((BT, K), lambda t: (t, 0)),
            pl.BlockSpec((1, 1), lambda t: (0, 0)),
        ],
        out_shape=[
            jax.ShapeDtypeStruct((N, 1), jnp.int32),
            jax.ShapeDtypeStruct((N, K), jnp.float32),
            jax.ShapeDtypeStruct((1, 1), jnp.float32),
        ],
    )(flat, e_ts, a, *bs)


def _make_sc_gather():
    info = plsc.get_sparse_core_info()
    nw = info.num_cores * info.num_subcores
    b_per_w = N // nw
    chunk = 128
    nchunk = b_per_w // chunk
    mesh = plsc.VectorSubcoreMesh(core_axis_name="c", subcore_axis_name="s")

    @functools.partial(
        pl.kernel, mesh=mesh,
        out_type=jax.ShapeDtypeStruct((N, D), jnp.float32),
        scratch_types=[
            pltpu.VMEM((chunk,), jnp.int32),
            pltpu.VMEM((chunk, D), jnp.float32),
            pltpu.SemaphoreType.DMA,
        ],
    )
    def gather_k(table_hbm, idx_hbm, out_hbm, idx_v, rows_v, sem):
        wid = lax.axis_index("s") * info.num_cores + lax.axis_index("c")
        base = wid * b_per_w
        for cidx in range(nchunk):
            off = base + cidx * chunk
            pltpu.sync_copy(idx_hbm.at[pl.ds(off, chunk)], idx_v)
            pltpu.async_copy(table_hbm.at[idx_v], rows_v, sem).wait()
            pltpu.sync_copy(rows_v, out_hbm.at[pl.ds(off, chunk)])

    return gather_k


def kernel(inputs, embedding):
    input_shape = inputs.shape
    flat = inputs.reshape(-1, D)
    a = jnp.sum(flat ** 2, axis=1, keepdims=True).reshape(N // BT, 1, BT)
    b = jnp.sum(embedding ** 2, axis=1)[None, :]
    bs = tuple(b[:, o:e] for o, e in CHUNKS)

    idx2d, encodings, lsum = _argmin_onehot(flat * 2.0, embedding, a, bs)

    gather_k = _make_sc_gather()
    quantized = gather_k(embedding, idx2d.reshape(-1))

    mse = lsum[0, 0] / jnp.float32(N * D)
    loss = HOLISTIC_COST * (mse + COMMITMENT_COST * mse)
    quantized_st = quantized.reshape(input_shape)
    return (loss, quantized_st, encodings, idx2d)


# speedup vs baseline: 1.0174x; 1.0102x over previous
import functools

import jax
import jax.numpy as jnp
from jax import lax
from jax.experimental import pallas as pl
from jax.experimental.pallas import tpu as pltpu
from jax.experimental.pallas import tpu_sc as plsc

K = 8192
D = 256
N = 8192
BT = 512
CHUNKS = ((0, 2048), (2048, 4096), (4096, 6144), (6144, 8192))
COMMITMENT_COST = 0.5
HOLISTIC_COST = 0.1


def _argmin_body(z2_ref, *refs):
    nch = len(CHUNKS)
    e_ref = refs[0]
    a_ref = refs[1]
    b_refs = refs[2:nch + 2]
    idx_ref, enc_ref, lsum_ref = refs[nch + 2:]
    t = pl.program_id(0)
    z2 = z2_ref[...]
    a = jnp.transpose(a_ref[0], (1, 0))
    w = CHUNKS[0][1] - CHUNKS[0][0]
    iota_s = lax.broadcasted_iota(jnp.int32, (BT, w), 1).astype(
        jnp.float32) * jnp.float32(2.0 ** -40)
    m_run = jnp.full((BT, 1), jnp.inf, jnp.float32)
    i_run = jnp.zeros((BT, 1), jnp.int32)
    for (off, end), b_ref in zip(CHUNKS, b_refs):
        c2 = lax.dot_general(z2, e_ref[off:end, :], (((1,), (1,)), ((), ())),
                             preferred_element_type=jnp.float32)
        d = (a + b_ref[...]) - c2
        cm = jnp.min(d, axis=1, keepdims=True)
        cand = (d - cm) + iota_s
        ci = (jnp.min(cand, axis=1, keepdims=True)
              * jnp.float32(2.0 ** 40)).astype(jnp.int32) + off
        upd = cm < m_run
        i_run = jnp.where(upd, ci, i_run)
        m_run = jnp.where(upd, cm, m_run).astype(jnp.bfloat16).astype(
            jnp.float32)
    idx_ref[...] = i_run
    col = lax.broadcasted_iota(jnp.int32, (BT, K), 1)
    enc_ref[...] = (col == i_run).astype(jnp.float32)
    s = jnp.sum(m_run, keepdims=True)

    @pl.when(t == 0)
    def _():
        lsum_ref[...] = s

    @pl.when(t != 0)
    def _():
        lsum_ref[...] += s


def _argmin_onehot(flat, e_ts, a, bs):
    grid = (N // BT,)
    widths = [e - o for o, e in CHUNKS]
    return pl.pallas_call(
        _argmin_body,
        grid=grid,
        in_specs=(
            [pl.BlockSpec((BT, D), lambda t: (t, 0)),
             pl.BlockSpec((K, D), lambda t: (0, 0)),
             pl.BlockSpec((1, 1, BT), lambda t: (t, 0, 0))]
            + [pl.BlockSpec((1, w), lambda t: (0, 0)) for w in widths]
        ),
        out_specs=[
            pl.BlockSpec((BT, 1), lambda t: (t, 0)),
            pl.BlockSpec((BT, K), lambda t: (t, 0)),
            pl.BlockSpec((1, 1), lambda t: (0, 0)),
        ],
        out_shape=[
            jax.ShapeDtypeStruct((N, 1), jnp.int32),
            jax.ShapeDtypeStruct((N, K), jnp.float32),
            jax.ShapeDtypeStruct((1, 1), jnp.float32),
        ],
    )(flat, e_ts, a, *bs)


def _make_sc_gather():
    info = plsc.get_sparse_core_info()
    nw = info.num_cores * info.num_subcores
    b_per_w = N // nw
    chunk = 128
    nchunk = b_per_w // chunk
    mesh = plsc.VectorSubcoreMesh(core_axis_name="c", subcore_axis_name="s")

    @functools.partial(
        pl.kernel, mesh=mesh,
        out_type=jax.ShapeDtypeStruct((N, D), jnp.float32),
        scratch_types=(
            [pltpu.VMEM((chunk,), jnp.int32) for _ in range(nchunk)]
            + [pltpu.VMEM((chunk, D), jnp.float32) for _ in range(nchunk)]
            + [pltpu.SemaphoreType.DMA for _ in range(nchunk)]
        ),
    )
    def gather_k(table_hbm, idx_hbm, out_hbm, *scratch):
        idx_vs = scratch[:nchunk]
        rows_vs = scratch[nchunk:2 * nchunk]
        sems = scratch[2 * nchunk:]
        wid = lax.axis_index("s") * info.num_cores + lax.axis_index("c")
        base = wid * b_per_w
        copies = []
        for cidx in range(nchunk):
            off = base + cidx * chunk
            pltpu.sync_copy(idx_hbm.at[pl.ds(off, chunk)], idx_vs[cidx])
            copies.append(pltpu.async_copy(
                table_hbm.at[idx_vs[cidx]], rows_vs[cidx], sems[cidx]))
        for cidx in range(nchunk):
            off = base + cidx * chunk
            copies[cidx].wait()
            pltpu.sync_copy(rows_vs[cidx], out_hbm.at[pl.ds(off, chunk)])

    return gather_k


def kernel(inputs, embedding):
    input_shape = inputs.shape
    flat = inputs.reshape(-1, D)
    a = jnp.sum(flat ** 2, axis=1, keepdims=True).reshape(N // BT, 1, BT)
    b = jnp.sum(embedding ** 2, axis=1)[None, :]
    bs = tuple(b[:, o:e] for o, e in CHUNKS)

    idx2d, encodings, lsum = _argmin_onehot(flat * 2.0, embedding, a, bs)

    gather_k = _make_sc_gather()
    quantized = gather_k(embedding, idx2d.reshape(-1))

    mse = lsum[0, 0] / jnp.float32(N * D)
    loss = HOLISTIC_COST * (mse + COMMITMENT_COST * mse)
    quantized_st = quantized.reshape(input_shape)
    return (loss, quantized_st, encodings, idx2d)


# speedup vs baseline: 1.0429x; 1.0250x over previous
import functools

import jax
import jax.numpy as jnp
from jax import lax
from jax.experimental import pallas as pl
from jax.experimental.pallas import tpu as pltpu
from jax.experimental.pallas import tpu_sc as plsc

K = 8192
D = 256
N = 8192
BT = 512
CHUNKS = ((0, 2048), (2048, 4096), (4096, 6144), (6144, 8192))
COMMITMENT_COST = 0.5
HOLISTIC_COST = 0.1


def _argmin_body(z2_ref, *refs):
    nch = len(CHUNKS)
    e_ref = refs[0]
    a_ref = refs[1]
    b_refs = refs[2:nch + 2]
    idx_ref, enc_ref, lsum_ref = refs[nch + 2:]
    t = pl.program_id(0)
    z2 = z2_ref[...]
    a = jnp.transpose(a_ref[0], (1, 0))
    w = CHUNKS[0][1] - CHUNKS[0][0]
    iota_s = lax.broadcasted_iota(jnp.int32, (BT, w), 1).astype(
        jnp.float32) * jnp.float32(2.0 ** -40)
    m_run = jnp.full((BT, 1), jnp.inf, jnp.float32)
    i_run = jnp.zeros((BT, 1), jnp.int32)
    for (off, end), b_ref in zip(CHUNKS, b_refs):
        c2 = lax.dot_general(z2, e_ref[off:end, :], (((1,), (1,)), ((), ())),
                             preferred_element_type=jnp.float32)
        d = (a + b_ref[...]) - c2
        cm = jnp.min(d, axis=1, keepdims=True)
        cand = (d - cm) + iota_s
        ci = (jnp.min(cand, axis=1, keepdims=True)
              * jnp.float32(2.0 ** 40)).astype(jnp.int32) + off
        upd = cm < m_run
        i_run = jnp.where(upd, ci, i_run)
        m_run = jnp.where(upd, cm, m_run).astype(jnp.bfloat16).astype(
            jnp.float32)
    idx_ref[...] = jnp.transpose(i_run, (1, 0)).reshape(1, 1, BT)
    col = lax.broadcasted_iota(jnp.int32, (BT, K), 1)
    enc_ref[...] = (col == i_run).astype(jnp.float32)
    s = jnp.sum(m_run, keepdims=True)

    @pl.when(t == 0)
    def _():
        lsum_ref[...] = s

    @pl.when(t != 0)
    def _():
        lsum_ref[...] += s


def _argmin_onehot(flat, e_ts, a, bs):
    grid = (N // BT,)
    widths = [e - o for o, e in CHUNKS]
    return pl.pallas_call(
        _argmin_body,
        grid=grid,
        in_specs=(
            [pl.BlockSpec((BT, D), lambda t: (t, 0)),
             pl.BlockSpec((K, D), lambda t: (0, 0)),
             pl.BlockSpec((1, 1, BT), lambda t: (t, 0, 0))]
            + [pl.BlockSpec((1, w), lambda t: (0, 0)) for w in widths]
        ),
        out_specs=[
            pl.BlockSpec((1, 1, BT), lambda t: (t, 0, 0)),
            pl.BlockSpec((BT, K), lambda t: (t, 0)),
            pl.BlockSpec((1, 1), lambda t: (0, 0)),
        ],
        out_shape=[
            jax.ShapeDtypeStruct((N // BT, 1, BT), jnp.int32),
            jax.ShapeDtypeStruct((N, K), jnp.float32),
            jax.ShapeDtypeStruct((1, 1), jnp.float32),
        ],
    )(flat, e_ts, a, *bs)


def _make_sc_gather():
    info = plsc.get_sparse_core_info()
    nw = info.num_cores * info.num_subcores
    b_per_w = N // nw
    chunk = 128
    nchunk = b_per_w // chunk
    mesh = plsc.VectorSubcoreMesh(core_axis_name="c", subcore_axis_name="s")

    @functools.partial(
        pl.kernel, mesh=mesh,
        out_type=jax.ShapeDtypeStruct((N, D), jnp.float32),
        scratch_types=(
            [pltpu.VMEM((chunk,), jnp.int32) for _ in range(nchunk)]
            + [pltpu.VMEM((chunk, D), jnp.float32) for _ in range(nchunk)]
            + [pltpu.SemaphoreType.DMA for _ in range(nchunk)]
        ),
    )
    def gather_k(table_hbm, idx_hbm, out_hbm, *scratch):
        idx_vs = scratch[:nchunk]
        rows_vs = scratch[nchunk:2 * nchunk]
        sems = scratch[2 * nchunk:]
        wid = lax.axis_index("s") * info.num_cores + lax.axis_index("c")
        base = wid * b_per_w
        copies = []
        for cidx in range(nchunk):
            off = base + cidx * chunk
            pltpu.sync_copy(idx_hbm.at[pl.ds(off, chunk)], idx_vs[cidx])
            copies.append(pltpu.async_copy(
                table_hbm.at[idx_vs[cidx]], rows_vs[cidx], sems[cidx]))
        for cidx in range(nchunk):
            off = base + cidx * chunk
            copies[cidx].wait()
            pltpu.sync_copy(rows_vs[cidx], out_hbm.at[pl.ds(off, chunk)])

    return gather_k


def kernel(inputs, embedding):
    input_shape = inputs.shape
    flat = inputs.reshape(-1, D)
    a = jnp.sum(flat ** 2, axis=1, keepdims=True).reshape(N // BT, 1, BT)
    b = jnp.sum(embedding ** 2, axis=1)[None, :]
    bs = tuple(b[:, o:e] for o, e in CHUNKS)

    idx3, encodings, lsum = _argmin_onehot(flat * 2.0, embedding, a, bs)
    idx2d = idx3.reshape(N, 1)

    gather_k = _make_sc_gather()
    quantized = gather_k(embedding, idx3.reshape(-1))

    mse = lsum[0, 0] / jnp.float32(N * D)
    loss = HOLISTIC_COST * (mse + COMMITMENT_COST * mse)
    quantized_st = quantized.reshape(input_shape)
    return (loss, quantized_st, encodings, idx2d)
